# Initial kernel scaffold; baseline (speedup 1.0000x reference)
import jax
import jax.numpy as jnp
from jax.experimental import pallas as pl


def kernel(index_sentences, W):
    raise NotImplementedError("write your pallas kernel here")


# speedup vs baseline: 21.3503x; 21.3503x over previous
import functools

import jax
import jax.numpy as jnp
from jax import lax
from jax.experimental import pallas as pl
from jax.experimental.pallas import tpu as pltpu
from jax.experimental.pallas import tpu_sc as plsc

_NF = 26
_V = 100
_D = 64
_B = 16384
_ROWS = _B * _NF
_NW = 32
_RPW = _ROWS // _NW
_CH = 512
_NCH = _RPW // _CH
_IDX_COLS = 128
_SUB = _CH // _IDX_COLS


def _sc_gather(idx2d, table):
    mesh = plsc.VectorSubcoreMesh(core_axis_name="c", subcore_axis_name="s")

    @functools.partial(
        pl.kernel,
        mesh=mesh,
        out_type=jax.ShapeDtypeStruct((_ROWS, _D), jnp.float32),
        scratch_types=[
            pltpu.VMEM((_SUB, _IDX_COLS), jnp.int32),
            pltpu.VMEM((_CH, _D), jnp.float32),
            pltpu.SemaphoreType.DMA,
        ],
        compiler_params=pltpu.CompilerParams(use_tc_tiling_on_sc=False),
    )
    def k(idx_hbm, table_hbm, out_hbm, idxbuf, rowsbuf, sem):
        w = lax.axis_index("s") * 2 + lax.axis_index("c")

        def chunk(c, carry):
            base = w * _RPW + c * _CH
            pltpu.sync_copy(
                idx_hbm.at[pl.ds(w * (_RPW // _IDX_COLS) + c * _SUB, _SUB)],
                idxbuf,
            )
            for r in range(_SUB):
                for g in range(_IDX_COLS // 16):
                    pos0 = base + r * _IDX_COLS + g * 16
                    p = lax.iota(jnp.int32, 16) + pos0
                    v = idxbuf[r, pl.ds(g * 16, 16)]
                    idxbuf[r, pl.ds(g * 16, 16)] = v + lax.rem(p, _NF) * _V
            copies = [
                pltpu.async_copy(
                    table_hbm.at[idxbuf.at[r]],
                    rowsbuf.at[pl.ds(r * _IDX_COLS, _IDX_COLS)],
                    sem,
                )
                for r in range(_SUB)
            ]
            for cp in copies:
                cp.wait()
            pltpu.sync_copy(rowsbuf, out_hbm.at[pl.ds(base, _CH)])
            return carry

        lax.fori_loop(0, _NCH, chunk, 0)

    return k(idx2d, table)


def kernel(index_sentences, W):
    idx2d = index_sentences.astype(jnp.int32).reshape(_ROWS // _IDX_COLS, _IDX_COLS)
    table = W.astype(jnp.float32).reshape(_NF * _V, _D)
    out = _sc_gather(idx2d, table)
    return out.reshape(_B, _NF * _D)


# speedup vs baseline: 22.9953x; 1.0770x over previous
import functools

import jax
import jax.numpy as jnp
from jax import lax
from jax.experimental import pallas as pl
from jax.experimental.pallas import tpu as pltpu
from jax.experimental.pallas import tpu_sc as plsc

_NF = 26
_V = 100
_D = 64
_B = 16384
_ROWS = _B * _NF
_NW = 32
_RPW = _ROWS // _NW
_CH = 512
_NCH = _RPW // _CH
_IDX_COLS = 128
_SUB = _CH // _IDX_COLS
_IDX_ROWS_W = _RPW // _IDX_COLS


def _sc_gather(idx2d, table):
    mesh = plsc.VectorSubcoreMesh(core_axis_name="c", subcore_axis_name="s")

    @functools.partial(
        pl.kernel,
        mesh=mesh,
        out_type=jax.ShapeDtypeStruct((_ROWS, _D), jnp.float32),
        scratch_types=[
            pltpu.VMEM((_IDX_ROWS_W, _IDX_COLS), jnp.int32),
            pltpu.VMEM((_CH, _D), jnp.float32),
            pltpu.VMEM((_CH, _D), jnp.float32),
            pltpu.SemaphoreType.DMA,
            pltpu.SemaphoreType.DMA,
        ],
        compiler_params=pltpu.CompilerParams(use_tc_tiling_on_sc=False),
    )
    def k(idx_hbm, table_hbm, out_hbm, idxall, rbuf0, rbuf1, sem0, sem1):
        w = lax.axis_index("s") * 2 + lax.axis_index("c")
        wbase = w * _RPW

        pltpu.sync_copy(idx_hbm.at[pl.ds(w * _IDX_ROWS_W, _IDX_ROWS_W)], idxall)

        def fire(c, rbuf, sem):
            for r in range(_SUB):
                row = c * _SUB + r
                for g in range(_IDX_COLS // 16):
                    pos0 = wbase + row * _IDX_COLS + g * 16
                    p = lax.iota(jnp.int32, 16) + pos0
                    v = idxall[row, pl.ds(g * 16, 16)]
                    idxall[row, pl.ds(g * 16, 16)] = v + lax.rem(p, _NF) * _V
            for r in range(_SUB):
                pltpu.async_copy(
                    table_hbm.at[idxall.at[c * _SUB + r]],
                    rbuf.at[pl.ds(r * _IDX_COLS, _IDX_COLS)],
                    sem,
                )

        def drain(rbuf, sem):
            pltpu.make_async_copy(out_hbm.at[pl.ds(0, _CH)], rbuf, sem).wait()

        fire(0, rbuf0, sem0)

        def pair(i, carry):
            a = 2 * i
            fire(a + 1, rbuf1, sem1)
            drain(rbuf0, sem0)
            pltpu.sync_copy(rbuf0, out_hbm.at[pl.ds(wbase + a * _CH, _CH)])

            @pl.when(i < _NCH // 2 - 1)
            def _():
                fire(a + 2, rbuf0, sem0)

            drain(rbuf1, sem1)
            pltpu.sync_copy(rbuf1, out_hbm.at[pl.ds(wbase + (a + 1) * _CH, _CH)])
            return carry

        lax.fori_loop(0, _NCH // 2, pair, 0)

    return k(idx2d, table)


def kernel(index_sentences, W):
    idx2d = index_sentences.astype(jnp.int32).reshape(_ROWS // _IDX_COLS, _IDX_COLS)
    table = W.astype(jnp.float32).reshape(_NF * _V, _D)
    out = _sc_gather(idx2d, table)
    return out.reshape(_B, _NF * _D)
